# baseline (speedup 1.0000x reference)
import jax
import jax.numpy as jnp
from jax import lax
from jax.experimental import pallas as pl
from jax.experimental.pallas import tpu as pltpu
from jax.experimental.pallas import tpu_sc as plsc

_N, _D = 4096, 2048
_NC, _L = 2, 16
_NSTRIPE = 16
_CPW = _D // _NSTRIPE
_VPR = _CPW // _L
_RB = 512


def _sc_body(x_hbm, out_hbm, buf, sem):
    c = lax.axis_index("c")
    s = lax.axis_index("s")
    wid = s * _NC + c

    @pl.when(s < _NSTRIPE // _NC)
    def _work():
        c0 = wid * _CPW

        def blk_step(blk, acc):
            pltpu.async_copy(
                x_hbm.at[pl.ds(blk * _RB, _RB), pl.ds(c0, _CPW)], buf, sem
            ).wait()

            def row_step(r, carry):
                new = []
                for j in range(_VPR):
                    a = carry[j] + buf[r, pl.ds(j * _L, _L)]
                    buf[r, pl.ds(j * _L, _L)] = a
                    new.append(a)
                return tuple(new)

            acc = lax.fori_loop(0, _RB, row_step, acc)

            pltpu.async_copy(
                buf, out_hbm.at[pl.ds(blk * _RB, _RB), pl.ds(c0, _CPW)], sem
            ).wait()
            return acc

        zeros = tuple(jnp.zeros((_L,), jnp.float32) for _ in range(_VPR))
        lax.fori_loop(0, _N // _RB, blk_step, zeros)


def kernel(x):
    mesh = plsc.VectorSubcoreMesh(core_axis_name="c", subcore_axis_name="s")
    run = pl.kernel(
        _sc_body,
        mesh=mesh,
        out_type=jax.ShapeDtypeStruct((_N, _D), jnp.float32),
        scratch_types=[
            pltpu.VMEM((_RB, _CPW), jnp.float32),
            pltpu.SemaphoreType.DMA,
        ],
    )
    return run(x)


# speedup vs baseline: 1.3736x; 1.3736x over previous
import jax
import jax.numpy as jnp
from jax import lax
from jax.experimental import pallas as pl
from jax.experimental.pallas import tpu as pltpu
from jax.experimental.pallas import tpu_sc as plsc

_N, _D = 4096, 2048
_NC, _L = 2, 16
_NSTRIPE = 16
_CPW = _D // _NSTRIPE
_VPR = _CPW // _L
_RB = 128
_NBUF = 4
_NBLK = _N // _RB


def _sc_body(x_hbm, out_hbm, *refs):
    bufs = refs[:_NBUF]
    sin = refs[_NBUF : 2 * _NBUF]
    sout = refs[2 * _NBUF : 3 * _NBUF]

    c = lax.axis_index("c")
    s = lax.axis_index("s")
    wid = s * _NC + c

    @pl.when(s < _NSTRIPE // _NC)
    def _work():
        c0 = wid * _CPW

        def in_slice(blk):
            return x_hbm.at[pl.ds(blk * _RB, _RB), pl.ds(c0, _CPW)]

        def out_slice(blk):
            return out_hbm.at[pl.ds(blk * _RB, _RB), pl.ds(c0, _CPW)]

        pltpu.async_copy(in_slice(0), bufs[0], sin[0])
        pltpu.async_copy(in_slice(1), bufs[1], sin[1])

        def ring_step(p, acc):
            for i in range(_NBUF):
                blk = p * _NBUF + i
                buf = bufs[i]
                pltpu.make_async_copy(in_slice(blk), buf, sin[i]).wait()

                def row_step(r, carry):
                    new = []
                    for j in range(_VPR):
                        a = carry[j] + buf[r, pl.ds(j * _L, _L)]
                        buf[r, pl.ds(j * _L, _L)] = a
                        new.append(a)
                    return tuple(new)

                acc = lax.fori_loop(0, _RB, row_step, acc, unroll=2)
                pltpu.async_copy(buf, out_slice(blk), sout[i])

                nxt = (i + 2) % _NBUF
                prev_blk = blk - 2

                @pl.when(prev_blk >= 0)
                def _drain():
                    pltpu.make_async_copy(
                        bufs[nxt], out_slice(prev_blk), sout[nxt]
                    ).wait()

                next_blk = blk + 2

                @pl.when(next_blk < _NBLK)
                def _refill():
                    pltpu.async_copy(in_slice(next_blk), bufs[nxt], sin[nxt])

            return acc

        zeros = tuple(jnp.zeros((_L,), jnp.float32) for _ in range(_VPR))
        lax.fori_loop(0, _NBLK // _NBUF, ring_step, zeros)

        for i in (_NBUF - 2, _NBUF - 1):
            pltpu.make_async_copy(
                bufs[i], out_slice(_NBLK - _NBUF + i), sout[i]
            ).wait()


def kernel(x):
    mesh = plsc.VectorSubcoreMesh(core_axis_name="c", subcore_axis_name="s")
    run = pl.kernel(
        _sc_body,
        mesh=mesh,
        out_type=jax.ShapeDtypeStruct((_N, _D), jnp.float32),
        scratch_types=(
            [pltpu.VMEM((_RB, _CPW), jnp.float32) for _ in range(_NBUF)]
            + [pltpu.SemaphoreType.DMA for _ in range(2 * _NBUF)]
        ),
    )
    return run(x)


# speedup vs baseline: 3.5506x; 2.5848x over previous
import jax
import jax.numpy as jnp
from jax.experimental import pallas as pl
from jax.experimental.pallas import tpu as pltpu

_ROWS_PER_BLOCK = 512
_SUB_ROWS = 16


def _cumsum_kern(x_ref, o_ref, carry_ref):
    i = pl.program_id(0)

    @pl.when(i == 0)
    def _zero_carry():
        carry_ref[...] = jnp.zeros_like(carry_ref)

    s = _SUB_ROWS
    tri = (
        jax.lax.broadcasted_iota(jnp.int32, (s, s), 0)
        >= jax.lax.broadcasted_iota(jnp.int32, (s, s), 1)
    ).astype(jnp.float32)
    carry = carry_ref[...]
    for b in range(_ROWS_PER_BLOCK // s):
        sub = x_ref[b * s : (b + 1) * s, :]
        local = jnp.dot(tri, sub, preferred_element_type=jnp.float32)
        o_ref[b * s : (b + 1) * s, :] = local + carry
        carry = carry + local[s - 1 : s, :]
    carry_ref[...] = carry


def kernel(x):
    n, d = x.shape
    r = _ROWS_PER_BLOCK
    return pl.pallas_call(
        _cumsum_kern,
        grid=(n // r,),
        in_specs=[pl.BlockSpec((r, d), lambda i: (i, 0))],
        out_specs=pl.BlockSpec((r, d), lambda i: (i, 0)),
        out_shape=jax.ShapeDtypeStruct((n, d), x.dtype),
        scratch_shapes=[pltpu.VMEM((1, d), jnp.float32)],
        compiler_params=pltpu.CompilerParams(
            dimension_semantics=("arbitrary",),
        ),
    )(x)


# speedup vs baseline: 3.5660x; 1.0043x over previous
import jax
import jax.numpy as jnp
from jax.experimental import pallas as pl
from jax.experimental.pallas import tpu as pltpu

_ROWS_PER_BLOCK = 512
_SUB_ROWS = 64


def _cumsum_kern(x_ref, o_ref, carry_ref):
    i = pl.program_id(0)

    @pl.when(i == 0)
    def _zero_carry():
        carry_ref[...] = jnp.zeros_like(carry_ref)

    s = _SUB_ROWS
    tri = (
        jax.lax.broadcasted_iota(jnp.int32, (s, s), 0)
        >= jax.lax.broadcasted_iota(jnp.int32, (s, s), 1)
    ).astype(jnp.float32)
    carry = carry_ref[...]
    for b in range(_ROWS_PER_BLOCK // s):
        sub = x_ref[b * s : (b + 1) * s, :]
        local = jnp.dot(tri, sub, preferred_element_type=jnp.float32)
        o_ref[b * s : (b + 1) * s, :] = local + carry
        carry = carry + local[s - 1 : s, :]
    carry_ref[...] = carry


def kernel(x):
    n, d = x.shape
    r = _ROWS_PER_BLOCK
    return pl.pallas_call(
        _cumsum_kern,
        grid=(n // r,),
        in_specs=[pl.BlockSpec((r, d), lambda i: (i, 0))],
        out_specs=pl.BlockSpec((r, d), lambda i: (i, 0)),
        out_shape=jax.ShapeDtypeStruct((n, d), x.dtype),
        scratch_shapes=[pltpu.VMEM((1, d), jnp.float32)],
        compiler_params=pltpu.CompilerParams(
            dimension_semantics=("arbitrary",),
        ),
    )(x)


# speedup vs baseline: 3.7784x; 1.0595x over previous
import jax
import jax.numpy as jnp
from jax.experimental import pallas as pl
from jax.experimental.pallas import tpu as pltpu

_ROWS_PER_BLOCK = 1024
_SUB_ROWS = 64


def _cumsum_kern(x_ref, o_ref, carry_ref):
    i = pl.program_id(0)

    @pl.when(i == 0)
    def _zero_carry():
        carry_ref[...] = jnp.zeros_like(carry_ref)

    s = _SUB_ROWS
    tri = (
        jax.lax.broadcasted_iota(jnp.int32, (s, s), 0)
        >= jax.lax.broadcasted_iota(jnp.int32, (s, s), 1)
    ).astype(jnp.float32)
    carry = carry_ref[...]
    for b in range(_ROWS_PER_BLOCK // s):
        sub = x_ref[b * s : (b + 1) * s, :]
        local = jnp.dot(tri, sub, preferred_element_type=jnp.float32)
        o_ref[b * s : (b + 1) * s, :] = local + carry
        carry = carry + local[s - 1 : s, :]
    carry_ref[...] = carry


def kernel(x):
    n, d = x.shape
    r = _ROWS_PER_BLOCK
    return pl.pallas_call(
        _cumsum_kern,
        grid=(n // r,),
        in_specs=[pl.BlockSpec((r, d), lambda i: (i, 0))],
        out_specs=pl.BlockSpec((r, d), lambda i: (i, 0)),
        out_shape=jax.ShapeDtypeStruct((n, d), x.dtype),
        scratch_shapes=[pltpu.VMEM((1, d), jnp.float32)],
        compiler_params=pltpu.CompilerParams(
            dimension_semantics=("arbitrary",),
        ),
    )(x)
